# baseline (speedup 1.0000x reference)
import functools

import jax
import jax.numpy as jnp
from jax import lax
from jax.experimental import pallas as pl
from jax.experimental.pallas import tpu as pltpu
from jax.experimental.pallas import tpu_sc as plsc

B = 16384
F = 32
TPR = 8
NC = 2
NS = 16
NW = NC * NS
BPW = B // NW
CHUNK = 128
NCH = BPW // CHUNK
W = 16
NWIN = BPW // W


def _mf_body(uid_hbm, iid_hbm, p_hbm, q_hbm, bu_hbm, bi_hbm, out_hbm,
             uid_v, iid_v, pu2, qi2, bu_v, bi_v, o_v, sem0, sem1, bsem):
    wid = lax.axis_index("s") * NC + lax.axis_index("c")
    base = wid * BPW
    row0 = wid * NCH
    sems = (sem0, sem1)

    pltpu.sync_copy(uid_hbm.at[pl.ds(row0, NCH)], uid_v)
    pltpu.sync_copy(iid_hbm.at[pl.ds(row0, NCH)], iid_v)

    bias_copies = []
    for j in range(NCH):
        dst = pl.ds(j * CHUNK, CHUNK)
        bias_copies.append(pltpu.make_async_copy(
            bu_hbm.at[uid_v.at[j]], bu_v.at[dst], bsem))
        bias_copies.append(pltpu.make_async_copy(
            bi_hbm.at[iid_v.at[j]], bi_v.at[dst], bsem))
    for c in bias_copies:
        c.start()

    def idx_vec(ref, w):
        return ref[w // TPR, pl.ds((w % TPR) * W, W)]

    def fire(w, slot):
        uvec = idx_vec(uid_v, w)
        ivec = idx_vec(iid_v, w)
        tu = lax.shift_right_logical(uvec, 3)
        ti = lax.shift_right_logical(ivec, 3)
        su = uvec & (TPR - 1)
        si = ivec & (TPR - 1)
        for k in range(W):
            pltpu.make_async_copy(
                p_hbm.at[tu[k]].at[su[k]], pu2.at[slot].at[k],
                sems[slot]).start()
            pltpu.make_async_copy(
                q_hbm.at[ti[k]].at[si[k]], qi2.at[slot].at[k],
                sems[slot]).start()

    def drain(w, slot):
        for k in range(W):
            pltpu.make_async_copy(
                p_hbm.at[0].at[0], pu2.at[slot].at[k], sems[slot]).wait()
            pltpu.make_async_copy(
                p_hbm.at[0].at[0], qi2.at[slot].at[k], sems[slot]).wait()

    fire(0, 0)
    fire(1, 1)
    for c in bias_copies:
        c.wait()

    lane = jnp.arange(W, dtype=jnp.int32)

    def compute(w, slot):
        pu = pu2.at[slot]
        qi = qi2.at[slot]
        acc = bu_v[pl.ds(w * W, W)] + bi_v[pl.ds(w * W, W)]
        for k in range(W):
            pk = plsc.bitcast(pu[k, pl.ds(0, F // 2)], jnp.bfloat16)
            qk = plsc.bitcast(qi[k, pl.ds(0, F // 2)], jnp.bfloat16)
            a0, a1 = plsc.unpack(pk, format=plsc.PackFormat.INTERLEAVED)
            b0, b1 = plsc.unpack(qk, format=plsc.PackFormat.INTERLEAVED)
            dot = jnp.sum(a0 * b0 + a1 * b1, axis=0)
            acc = jnp.where(lane == k, acc + dot, acc)
        o_v[pl.ds(w * W, W)] = acc

    def body2(h, _):
        for b in range(2):
            w = 2 * h + b
            drain(w, b)
            compute(w, b)

            @pl.when(w + 2 < NWIN)
            def _():
                fire(w + 2, b)
        return ()

    lax.fori_loop(0, NWIN // 2, body2, ())

    pltpu.sync_copy(o_v, out_hbm.at[pl.ds(base, BPW)])


def kernel(user_id, item_id, P, Q, user_bias, item_bias):
    uid = user_id.astype(jnp.int32).reshape(NW * NCH, CHUNK)
    iid = item_id.astype(jnp.int32).reshape(NW * NCH, CHUNK)
    p3 = lax.bitcast_convert_type(
        P.reshape(P.shape[0] // TPR, TPR, F // 2, 2).astype(jnp.bfloat16),
        jnp.int32)
    q3 = lax.bitcast_convert_type(
        Q.reshape(Q.shape[0] // TPR, TPR, F // 2, 2).astype(jnp.bfloat16),
        jnp.int32)

    mesh = plsc.VectorSubcoreMesh(core_axis_name="c", subcore_axis_name="s")
    mf = functools.partial(
        pl.kernel,
        mesh=mesh,
        compiler_params=pltpu.CompilerParams(needs_layout_passes=False),
        out_type=jax.ShapeDtypeStruct((B,), jnp.float32),
        scratch_types=[
            pltpu.VMEM((NCH, CHUNK), jnp.int32),
            pltpu.VMEM((NCH, CHUNK), jnp.int32),
            pltpu.VMEM((2, W, F // 2), jnp.int32),
            pltpu.VMEM((2, W, F // 2), jnp.int32),
            pltpu.VMEM((BPW,), jnp.float32),
            pltpu.VMEM((BPW,), jnp.float32),
            pltpu.VMEM((BPW,), jnp.float32),
            pltpu.SemaphoreType.DMA,
            pltpu.SemaphoreType.DMA,
            pltpu.SemaphoreType.DMA,
        ],
    )(_mf_body)
    return mf(uid, iid, p3, q3,
              user_bias.reshape(-1), item_bias.reshape(-1))


# speedup vs baseline: 20.3701x; 20.3701x over previous
import functools

import jax
import jax.numpy as jnp
from jax import lax
from jax.experimental import pallas as pl
from jax.experimental.pallas import tpu as pltpu
from jax.experimental.pallas import tpu_sc as plsc

B = 16384
F = 32
TPR = 8
NC = 2
NS = 16
NW = NC * NS
BPW = B // NW
CHUNK = 128
NCH = BPW // CHUNK
W = 16
NWIN = BPW // W


def _mf_body(uid_hbm, iid_hbm, p_hbm, q_hbm, bu_hbm, bi_hbm, out_hbm,
             uid_v, iid_v, pu2, qi2, bu_v, bi_v, o_v, sem0, sem1, bsem):
    wid = lax.axis_index("s") * NC + lax.axis_index("c")
    base = wid * BPW
    row0 = wid * NCH
    sems = (sem0, sem1)

    pltpu.sync_copy(uid_hbm.at[pl.ds(row0, NCH)], uid_v)
    pltpu.sync_copy(iid_hbm.at[pl.ds(row0, NCH)], iid_v)

    bias_copies = []
    for j in range(NCH):
        dst = pl.ds(j * CHUNK, CHUNK)
        bias_copies.append(pltpu.make_async_copy(
            bu_hbm.at[uid_v.at[j]], bu_v.at[dst], bsem))
        bias_copies.append(pltpu.make_async_copy(
            bi_hbm.at[iid_v.at[j]], bi_v.at[dst], bsem))
    for c in bias_copies:
        c.start()

    def idx_vec(ref, w):
        return ref[w // TPR, pl.ds((w % TPR) * W, W)]

    def fire(w, slot):
        uvec = idx_vec(uid_v, w)
        ivec = idx_vec(iid_v, w)
        tu = lax.shift_right_logical(uvec, 3)
        ti = lax.shift_right_logical(ivec, 3)
        su = uvec & (TPR - 1)
        si = ivec & (TPR - 1)
        for k in range(W):
            pltpu.make_async_copy(
                p_hbm.at[tu[k]].at[su[k]], pu2.at[slot].at[k],
                sems[slot]).start()
            pltpu.make_async_copy(
                q_hbm.at[ti[k]].at[si[k]], qi2.at[slot].at[k],
                sems[slot]).start()

    def drain(w, slot):
        for k in range(W):
            pltpu.make_async_copy(
                p_hbm.at[0].at[0], pu2.at[slot].at[k], sems[slot]).wait()
            pltpu.make_async_copy(
                p_hbm.at[0].at[0], qi2.at[slot].at[k], sems[slot]).wait()

    fire(0, 0)
    fire(1, 1)
    for c in bias_copies:
        c.wait()

    lane = jnp.arange(W, dtype=jnp.int32)

    def compute(w, slot):
        pu = pu2.at[slot]
        qi = qi2.at[slot]
        acc = bu_v[pl.ds(w * W, W)] + bi_v[pl.ds(w * W, W)]
        for k in range(W):
            a0 = pu[k, pl.ds(0, 16)]
            a1 = pu[k, pl.ds(16, 16)]
            b0 = qi[k, pl.ds(0, 16)]
            b1 = qi[k, pl.ds(16, 16)]
            dot = jnp.sum(a0 * b0 + a1 * b1, axis=0)
            acc = jnp.where(lane == k, acc + dot, acc)
        o_v[pl.ds(w * W, W)] = acc

    def body2(h, _):
        for b in range(2):
            w = 2 * h + b
            drain(w, b)
            compute(w, b)

            @pl.when(w + 2 < NWIN)
            def _():
                fire(w + 2, b)
        return ()

    lax.fori_loop(0, NWIN // 2, body2, ())

    pltpu.sync_copy(o_v, out_hbm.at[pl.ds(base, BPW)])


def kernel(user_id, item_id, P, Q, user_bias, item_bias):
    uid = user_id.astype(jnp.int32).reshape(NW * NCH, CHUNK)
    iid = item_id.astype(jnp.int32).reshape(NW * NCH, CHUNK)
    p3 = P.reshape(P.shape[0] // TPR, TPR, F)
    q3 = Q.reshape(Q.shape[0] // TPR, TPR, F)

    mesh = plsc.VectorSubcoreMesh(core_axis_name="c", subcore_axis_name="s")
    mf = functools.partial(
        pl.kernel,
        mesh=mesh,
        compiler_params=pltpu.CompilerParams(needs_layout_passes=False),
        out_type=jax.ShapeDtypeStruct((B,), jnp.float32),
        scratch_types=[
            pltpu.VMEM((NCH, CHUNK), jnp.int32),
            pltpu.VMEM((NCH, CHUNK), jnp.int32),
            pltpu.VMEM((2, W, F), jnp.float32),
            pltpu.VMEM((2, W, F), jnp.float32),
            pltpu.VMEM((BPW,), jnp.float32),
            pltpu.VMEM((BPW,), jnp.float32),
            pltpu.VMEM((BPW,), jnp.float32),
            pltpu.SemaphoreType.DMA,
            pltpu.SemaphoreType.DMA,
            pltpu.SemaphoreType.DMA,
        ],
    )(_mf_body)
    return mf(uid, iid, p3, q3,
              user_bias.reshape(-1), item_bias.reshape(-1))
